# baseline (speedup 1.0000x reference)
import functools

import jax
import jax.numpy as jnp
from jax import lax
from jax.experimental import pallas as pl
from jax.experimental.pallas import tpu as pltpu
from jax.experimental.pallas import tpu_sc as plsc

G = 128
N = G * G
HID = 512
STEPS = 100
L = 16
OUTPAD = 128
BIGF = 3.0e38
NEGF = -3.0e38
f32 = jnp.float32
i32 = jnp.int32


_BLK = 4096
_NBLK = N // _BLK


def _actor_body(x_ref, ws_ref, wd_ref, b_ref, v_ref,
                s0_ref, s1_ref, st0_ref, m_sc, s_sc, p_sc):
    i = pl.program_id(0)
    z = lax.dot_general(ws_ref[...], x_ref[...], (((1,), (0,)), ((), ())),
                        preferred_element_type=f32)
    h0 = jnp.tanh(z + b_ref[...])
    s0 = lax.dot_general(v_ref[...], h0, (((1,), (0,)), ((), ())),
                         preferred_element_type=f32)
    s0_ref[...] = s0
    h1 = jnp.tanh((z + wd_ref[...]) + b_ref[...])
    s1_ref[...] = lax.dot_general(v_ref[...], h1, (((1,), (0,)), ((), ())),
                                  preferred_element_type=f32)
    q = s0 + f32(10000.0)
    bm = jnp.max(q)
    idx = lax.broadcasted_iota(i32, (1, _BLK), 1).astype(f32) + (
        i * _BLK).astype(f32)
    bi = jnp.min(jnp.where(q == bm, idx, BIGF))
    bS = jnp.sum(jnp.exp(q - bm))

    @pl.when(i == 0)
    def _():
        m_sc[0] = bm
        p_sc[0] = bi
        s_sc[0] = bS

    @pl.when(i > 0)
    def _():
        m_old = m_sc[0]
        m_new = jnp.maximum(m_old, bm)
        s_sc[0] = s_sc[0] * jnp.exp(m_old - m_new) + bS * jnp.exp(bm - m_new)
        p_sc[0] = jnp.where(
            bm > m_old, bi,
            jnp.where(bm == m_old, jnp.minimum(p_sc[0], bi), p_sc[0]))
        m_sc[0] = m_new

    @pl.when(i == _NBLK - 1)
    def _():
        ii = lax.broadcasted_iota(i32, (1, L), 1)
        st0_ref[...] = jnp.where(
            ii == 0, s_sc[0], jnp.where(ii == 1, p_sc[0], f32(0.0)))


def _actor_scores(static2, W_s, wd_col, b_col, v_row):
    return pl.pallas_call(
        _actor_body,
        grid=(_NBLK,),
        in_specs=[
            pl.BlockSpec((2, _BLK), lambda i: (0, i)),
            pl.BlockSpec((HID, 2), lambda i: (0, 0)),
            pl.BlockSpec((HID, 1), lambda i: (0, 0)),
            pl.BlockSpec((HID, 1), lambda i: (0, 0)),
            pl.BlockSpec((1, HID), lambda i: (0, 0)),
        ],
        out_specs=[
            pl.BlockSpec((1, _BLK), lambda i: (0, i)),
            pl.BlockSpec((1, _BLK), lambda i: (0, i)),
            pl.BlockSpec((1, L), lambda i: (0, 0)),
        ],
        out_shape=[
            jax.ShapeDtypeStruct((1, N), f32),
            jax.ShapeDtypeStruct((1, N), f32),
            jax.ShapeDtypeStruct((1, L), f32),
        ],
        scratch_shapes=[
            pltpu.SMEM((1,), f32),
            pltpu.SMEM((1,), f32),
            pltpu.SMEM((1,), f32),
        ],
    )(static2, W_s, wd_col, b_col, v_row)


def _dir_rows(lanes):
    def dr_of(d):
        return jnp.where(d < 3, -1, jnp.where(d < 5, 0, 1)).astype(i32)

    def dc_of(d):
        return jnp.where(
            d < 3, d - 1,
            jnp.where(d == 3, -1, jnp.where(d == 4, 1, d - 6))).astype(i32)

    lo = lanes < 8
    dr8 = jnp.where(lo, dr_of(lanes), 99)
    dc8 = jnp.where(lo, dc_of(lanes), 99)
    hi = lanes >= 8
    dh = lanes - 8
    drh = jnp.where(hi, dr_of(dh), 99)
    dch = jnp.where(hi, dc_of(dh), 99)
    return dr8, dc8, drh, dch


def _newton_log(S):
    bits = plsc.bitcast(S, i32)
    e = ((bits >> 23) & 0xFF) - 127
    mant = plsc.bitcast((bits & 0x7FFFFF) | 0x3F800000, f32)
    t = mant - 1.0
    y = e.astype(f32) * f32(0.6931471805599453) + t * (1.0 + t * (-0.5 + t * f32(0.3333333)))
    y = y + S * jnp.exp(-y) - 1.0
    y = y + S * jnp.exp(-y) - 1.0
    return y


def _take_lane(x, k):
    return jnp.take_along_axis(
        x, jnp.full((L,), k, i32), axis=0, mode="promise_in_bounds")


def _decode_body(s0_hbm, s1_hbm, st0_hbm, lim_hbm, ptr_hbm, logp_hbm,
                 cur_v, s1_v, ptr_v, sbuf_v, logp_v, st0_v, lim_v):
    cid = lax.axis_index("c")
    sid = lax.axis_index("s")

    @pl.when(jnp.logical_and(cid == 0, sid == 0))
    def _():
        pltpu.sync_copy(s0_hbm, cur_v)
        pltpu.sync_copy(s1_hbm, s1_v)
        pltpu.sync_copy(st0_hbm, st0_v)
        pltpu.sync_copy(lim_hbm, lim_v)
        lanes = lax.iota(i32, L)
        lanes_f = lanes.astype(f32)
        lim_vec = lim_v[...]
        dr8, dc8, drh, dch = _dir_rows(lanes)
        lane0 = lanes == 0
        zero_v = jnp.zeros((L,), i32)

        def update(t, t_vec, ptr_vec, S_vec, dv, allow, flat, lr, lc):
            active = t_vec < lim_vec
            plsc.store_scatter(ptr_v, [t_vec], ptr_vec, mask=lane0)
            plsc.store_scatter(sbuf_v, [t_vec], S_vec, mask=lane0)
            r = ptr_vec >> 7
            c = ptr_vec & 127
            is0 = t_vec == 0
            lr_eff = jnp.where(is0, r, lr)
            lc_eff = jnp.where(is0, c, lc)
            sgr = jnp.sign(r - lr_eff)
            sgc = jnp.sign(c - lc_eff)
            moved = jnp.logical_or(sgr != 0, sgc != 0)
            match = jnp.logical_and(drh == sgr, dch == sgc)
            opp = lax.rev(match.astype(i32), (0,))
            dv_new = dv | jnp.where(
                jnp.logical_and(moved, active), opp, zero_v)
            nr = r + dr8
            nc = c + dc8
            inb = (nr >= 0) & (nr < G) & (nc >= 0) & (nc < G)
            allow_new = jnp.logical_and(dv_new == 0, inb).astype(i32)
            flat_new = jnp.clip(nr, 0, G - 1) * G + jnp.clip(nc, 0, G - 1)
            s1p = plsc.load_gather(s1_v, [ptr_vec])
            plsc.store_scatter(cur_v, [ptr_vec], s1p,
                               mask=jnp.logical_and(lane0, active))
            dv = jnp.where(active, dv_new, dv)
            allow = jnp.where(active, allow_new, allow)
            flat = jnp.where(active, flat_new, flat)
            lr = jnp.where(active, r, lr_eff)
            lc = jnp.where(active, c, lc_eff)
            return dv, allow, flat, lr, lc

        def dense_scan():
            def p1(i, mx):
                return jnp.maximum(mx, cur_v[pl.ds(i * L, L)])

            mx = lax.fori_loop(0, N // L, p1, jnp.full((L,), NEGF, f32))
            m_vec = _take_lane(plsc.cummax(mx), L - 1)

            def p2(i, carry):
                sacc, imin = carry
                q = cur_v[pl.ds(i * L, L)]
                sacc = sacc + jnp.exp(q - m_vec)
                idx = lanes_f + jnp.broadcast_to((i * L).astype(f32), (L,))
                imin = jnp.minimum(imin, jnp.where(q == m_vec, idx, BIGF))
                return sacc, imin

            sacc, imin = lax.fori_loop(
                0, N // L, p2,
                (jnp.zeros((L,), f32), jnp.full((L,), BIGF, f32)))
            S_vec = _take_lane(plsc.cumsum(sacc), L - 1)
            ptr_vec = (-_take_lane(plsc.cummax(-imin), L - 1)).astype(i32)
            return S_vec, ptr_vec

        def sparse_scan(flat_vec, allow_vec):
            allow_b = allow_vec != 0
            g = plsc.load_gather(cur_v, [flat_vec])
            qm = jnp.where(allow_b, g + f32(10000.0), NEGF)
            vals = plsc.sort_key_val(qm, flat_vec, descending=True)
            keys, ptrs = vals[0], vals[1]
            m_vec = _take_lane(keys, 0)
            ptr_vec = _take_lane(ptrs, 0)
            e = jnp.where(allow_b, jnp.exp(qm - m_vec), f32(0.0))
            S_vec = lax.rev(plsc.cumsum(e), (0,))
            return S_vec, ptr_vec

        st0 = st0_v[...]
        S0_vec = _take_lane(st0, 0)
        ptr0_vec = _take_lane(st0, 1).astype(i32)
        carry0 = update(0, jnp.zeros((L,), i32), ptr0_vec, S0_vec,
                        zero_v, zero_v, zero_v, zero_v, zero_v)

        def step(t, carry):
            dv, allow, flat, lr, lc = carry
            t_vec = jnp.broadcast_to(t, (L,))
            no_allow = jnp.max(allow.astype(f32)) == f32(0.0)
            S_vec, ptr_vec = lax.cond(
                no_allow,
                lambda fv, av: dense_scan(),
                sparse_scan,
                flat, allow)
            return update(t, t_vec, ptr_vec, S_vec, dv, allow, flat, lr, lc)

        lax.fori_loop(1, STEPS, step, carry0, unroll=3)

        for i in range(OUTPAD // L):
            sl = pl.ds(i * L, L)
            S = jnp.maximum(sbuf_v[sl], f32(1.0))
            logp_v[sl] = -_newton_log(S)

        pltpu.sync_copy(ptr_v.at[pl.ds(0, STEPS)], ptr_hbm)
        pltpu.sync_copy(logp_v.at[pl.ds(0, STEPS)], logp_hbm)


@functools.partial(
    pl.kernel,
    out_type=[
        jax.ShapeDtypeStruct((STEPS,), i32),
        jax.ShapeDtypeStruct((STEPS,), f32),
    ],
    mesh=plsc.VectorSubcoreMesh(core_axis_name="c", subcore_axis_name="s"),
    compiler_params=pltpu.CompilerParams(needs_layout_passes=False),
    scratch_types=[
        pltpu.VMEM((N,), f32),
        pltpu.VMEM((N,), f32),
        pltpu.VMEM((OUTPAD,), i32),
        pltpu.VMEM((OUTPAD,), f32),
        pltpu.VMEM((OUTPAD,), f32),
        pltpu.VMEM((L,), f32),
        pltpu.VMEM((L,), i32),
    ],
)
def _decode(s0_hbm, s1_hbm, st0_hbm, lim_hbm, ptr_hbm, logp_hbm, *scratch):
    _decode_body(s0_hbm, s1_hbm, st0_hbm, lim_hbm, ptr_hbm, logp_hbm, *scratch)


def kernel(static, dynamic, W_s, W_d, b, v, station_num_lim):
    del dynamic
    static2 = static[0]
    wd_col = W_d.astype(f32)
    b_col = b[:, None]
    v_row = v[None, :]
    s0, s1, st0 = _actor_scores(static2, W_s, wd_col, b_col, v_row)
    lim16 = jnp.broadcast_to(
        jnp.asarray(station_num_lim, i32), (L,))
    ptr100, logp100 = _decode(s0[0], s1[0], st0[0], lim16)
    return ptr100[None, :], logp100[None, :]


# speedup vs baseline: 1.0269x; 1.0269x over previous
import functools

import jax
import jax.numpy as jnp
from jax import lax
from jax.experimental import pallas as pl
from jax.experimental.pallas import tpu as pltpu
from jax.experimental.pallas import tpu_sc as plsc

G = 128
N = G * G
HID = 512
STEPS = 100
L = 16
OUTPAD = 128
BIGF = 3.0e38
NEGF = -3.0e38
f32 = jnp.float32
i32 = jnp.int32


_BLK = 4096
_NBLK = N // _BLK


def _actor_body(x_ref, ws_ref, wd_ref, b_ref, v_ref, lim_ref,
                s0_ref, s1_ref, st0_ref, m_sc, s_sc, p_sc):
    i = pl.program_id(0)
    z = lax.dot_general(ws_ref[...], x_ref[...], (((1,), (0,)), ((), ())),
                        preferred_element_type=f32)
    h0 = jnp.tanh(z + b_ref[...])
    s0 = lax.dot_general(v_ref[...], h0, (((1,), (0,)), ((), ())),
                         preferred_element_type=f32)
    s0_ref[...] = s0
    h1 = jnp.tanh((z + wd_ref[...]) + b_ref[...])
    s1_ref[...] = lax.dot_general(v_ref[...], h1, (((1,), (0,)), ((), ())),
                                  preferred_element_type=f32)
    q = s0 + f32(10000.0)
    bm = jnp.max(q)
    idx = lax.broadcasted_iota(i32, (1, _BLK), 1).astype(f32) + (
        i * _BLK).astype(f32)
    bi = jnp.min(jnp.where(q == bm, idx, BIGF))
    bS = jnp.sum(jnp.exp(q - bm))

    @pl.when(i == 0)
    def _():
        m_sc[0] = bm
        p_sc[0] = bi
        s_sc[0] = bS

    @pl.when(i > 0)
    def _():
        m_old = m_sc[0]
        m_new = jnp.maximum(m_old, bm)
        s_sc[0] = s_sc[0] * jnp.exp(m_old - m_new) + bS * jnp.exp(bm - m_new)
        p_sc[0] = jnp.where(
            bm > m_old, bi,
            jnp.where(bm == m_old, jnp.minimum(p_sc[0], bi), p_sc[0]))
        m_sc[0] = m_new

    @pl.when(i == _NBLK - 1)
    def _():
        ii = lax.broadcasted_iota(i32, (1, L), 1)
        limf = jnp.clip(lim_ref[0], -1, 127).astype(f32)
        st0_ref[...] = jnp.where(
            ii == 0, s_sc[0],
            jnp.where(ii == 1, p_sc[0], jnp.where(ii == 2, limf, f32(0.0))))


def _actor_scores(static2, W_s, wd_col, b_col, v_row, lim1):
    return pl.pallas_call(
        _actor_body,
        grid=(_NBLK,),
        in_specs=[
            pl.BlockSpec((2, _BLK), lambda i: (0, i)),
            pl.BlockSpec((HID, 2), lambda i: (0, 0)),
            pl.BlockSpec((HID, 1), lambda i: (0, 0)),
            pl.BlockSpec((HID, 1), lambda i: (0, 0)),
            pl.BlockSpec((1, HID), lambda i: (0, 0)),
            pl.BlockSpec(memory_space=pltpu.SMEM),
        ],
        out_specs=[
            pl.BlockSpec((1, _BLK), lambda i: (0, i)),
            pl.BlockSpec((1, _BLK), lambda i: (0, i)),
            pl.BlockSpec((1, L), lambda i: (0, 0)),
        ],
        out_shape=[
            jax.ShapeDtypeStruct((1, N), f32),
            jax.ShapeDtypeStruct((1, N), f32),
            jax.ShapeDtypeStruct((1, L), f32),
        ],
        scratch_shapes=[
            pltpu.SMEM((1,), f32),
            pltpu.SMEM((1,), f32),
            pltpu.SMEM((1,), f32),
        ],
    )(static2, W_s, wd_col, b_col, v_row, lim1)


def _dir_rows(lanes):
    def dr_of(d):
        return jnp.where(d < 3, -1, jnp.where(d < 5, 0, 1)).astype(i32)

    def dc_of(d):
        return jnp.where(
            d < 3, d - 1,
            jnp.where(d == 3, -1, jnp.where(d == 4, 1, d - 6))).astype(i32)

    lo = lanes < 8
    dr8 = jnp.where(lo, dr_of(lanes), 99)
    dc8 = jnp.where(lo, dc_of(lanes), 99)
    hi = lanes >= 8
    dh = lanes - 8
    drh = jnp.where(hi, dr_of(dh), 99)
    dch = jnp.where(hi, dc_of(dh), 99)
    return dr8, dc8, drh, dch


def _newton_log(S):
    bits = plsc.bitcast(S, i32)
    e = ((bits >> 23) & 0xFF) - 127
    mant = plsc.bitcast((bits & 0x7FFFFF) | 0x3F800000, f32)
    t = mant - 1.0
    y = e.astype(f32) * f32(0.6931471805599453) + t * (1.0 + t * (-0.5 + t * f32(0.3333333)))
    y = y + S * jnp.exp(-y) - 1.0
    y = y + S * jnp.exp(-y) - 1.0
    return y


def _take_lane(x, k):
    return jnp.take_along_axis(
        x, jnp.full((L,), k, i32), axis=0, mode="promise_in_bounds")


def _decode_body(s0_hbm, s1_hbm, st0_hbm, ptr_hbm, logp_hbm,
                 cur_v, s1_v, ptr_v, sbuf_v, logp_v, st0_v):
    cid = lax.axis_index("c")
    sid = lax.axis_index("s")

    @pl.when(jnp.logical_and(cid == 0, sid == 0))
    def _():
        pltpu.sync_copy(s0_hbm, cur_v)
        pltpu.sync_copy(s1_hbm, s1_v)
        pltpu.sync_copy(st0_hbm, st0_v)
        lanes = lax.iota(i32, L)
        lanes_f = lanes.astype(f32)
        dr8, dc8, drh, dch = _dir_rows(lanes)
        lane0 = lanes == 0
        zero_v = jnp.zeros((L,), i32)

        def update(t, t_vec, ptr_vec, S_vec, dv, allow, flat, lr, lc):
            active = t_vec < lim_vec
            plsc.store_scatter(ptr_v, [t_vec], ptr_vec, mask=lane0)
            plsc.store_scatter(sbuf_v, [t_vec], S_vec, mask=lane0)
            r = ptr_vec >> 7
            c = ptr_vec & 127
            is0 = t_vec == 0
            lr_eff = jnp.where(is0, r, lr)
            lc_eff = jnp.where(is0, c, lc)
            sgr = jnp.sign(r - lr_eff)
            sgc = jnp.sign(c - lc_eff)
            moved = jnp.logical_or(sgr != 0, sgc != 0)
            match = jnp.logical_and(drh == sgr, dch == sgc)
            opp = lax.rev(match.astype(i32), (0,))
            dv_new = dv | jnp.where(
                jnp.logical_and(moved, active), opp, zero_v)
            nr = r + dr8
            nc = c + dc8
            inb = (nr >= 0) & (nr < G) & (nc >= 0) & (nc < G)
            allow_new = jnp.logical_and(dv_new == 0, inb).astype(i32)
            flat_new = jnp.clip(nr, 0, G - 1) * G + jnp.clip(nc, 0, G - 1)
            s1p = plsc.load_gather(s1_v, [ptr_vec])
            plsc.store_scatter(cur_v, [ptr_vec], s1p,
                               mask=jnp.logical_and(lane0, active))
            dv = jnp.where(active, dv_new, dv)
            allow = jnp.where(active, allow_new, allow)
            flat = jnp.where(active, flat_new, flat)
            lr = jnp.where(active, r, lr_eff)
            lc = jnp.where(active, c, lc_eff)
            return dv, allow, flat, lr, lc

        def dense_scan():
            def p1(i, mx):
                return jnp.maximum(mx, cur_v[pl.ds(i * L, L)])

            mx = lax.fori_loop(0, N // L, p1, jnp.full((L,), NEGF, f32))
            m_vec = _take_lane(plsc.cummax(mx), L - 1)

            def p2(i, carry):
                sacc, imin = carry
                q = cur_v[pl.ds(i * L, L)]
                sacc = sacc + jnp.exp(q - m_vec)
                idx = lanes_f + jnp.broadcast_to((i * L).astype(f32), (L,))
                imin = jnp.minimum(imin, jnp.where(q == m_vec, idx, BIGF))
                return sacc, imin

            sacc, imin = lax.fori_loop(
                0, N // L, p2,
                (jnp.zeros((L,), f32), jnp.full((L,), BIGF, f32)))
            S_vec = _take_lane(plsc.cumsum(sacc), L - 1)
            ptr_vec = (-_take_lane(plsc.cummax(-imin), L - 1)).astype(i32)
            return S_vec, ptr_vec

        def sparse_scan(flat_vec, allow_vec):
            allow_b = allow_vec != 0
            g = plsc.load_gather(cur_v, [flat_vec])
            qm = jnp.where(allow_b, g + f32(10000.0), NEGF)
            vals = plsc.sort_key_val(qm, flat_vec, descending=True)
            keys, ptrs = vals[0], vals[1]
            m_vec = _take_lane(keys, 0)
            ptr_vec = _take_lane(ptrs, 0)
            e = jnp.where(allow_b, jnp.exp(qm - m_vec), f32(0.0))
            S_vec = lax.rev(plsc.cumsum(e), (0,))
            return S_vec, ptr_vec

        st0 = st0_v[...]
        S0_vec = _take_lane(st0, 0)
        ptr0_vec = _take_lane(st0, 1).astype(i32)
        lim_vec = _take_lane(st0, 2).astype(i32)
        carry0 = update(0, jnp.zeros((L,), i32), ptr0_vec, S0_vec,
                        zero_v, zero_v, zero_v, zero_v, zero_v)

        def step(t, carry):
            dv, allow, flat, lr, lc = carry
            t_vec = jnp.broadcast_to(t, (L,))
            no_allow = jnp.max(allow.astype(f32)) == f32(0.0)
            S_vec, ptr_vec = lax.cond(
                no_allow,
                lambda fv, av: dense_scan(),
                sparse_scan,
                flat, allow)
            return update(t, t_vec, ptr_vec, S_vec, dv, allow, flat, lr, lc)

        lax.fori_loop(1, STEPS, step, carry0)

        for i in range(OUTPAD // L):
            sl = pl.ds(i * L, L)
            S = jnp.maximum(sbuf_v[sl], f32(1.0))
            logp_v[sl] = -_newton_log(S)

        pltpu.sync_copy(ptr_v.at[pl.ds(0, STEPS)], ptr_hbm)
        pltpu.sync_copy(logp_v.at[pl.ds(0, STEPS)], logp_hbm)


@functools.partial(
    pl.kernel,
    out_type=[
        jax.ShapeDtypeStruct((STEPS,), i32),
        jax.ShapeDtypeStruct((STEPS,), f32),
    ],
    mesh=plsc.VectorSubcoreMesh(core_axis_name="c", subcore_axis_name="s"),
    compiler_params=pltpu.CompilerParams(needs_layout_passes=False),
    scratch_types=[
        pltpu.VMEM((N,), f32),
        pltpu.VMEM((N,), f32),
        pltpu.VMEM((OUTPAD,), i32),
        pltpu.VMEM((OUTPAD,), f32),
        pltpu.VMEM((OUTPAD,), f32),
        pltpu.VMEM((L,), f32),
    ],
)
def _decode(s0_hbm, s1_hbm, st0_hbm, ptr_hbm, logp_hbm, *scratch):
    _decode_body(s0_hbm, s1_hbm, st0_hbm, ptr_hbm, logp_hbm, *scratch)


def kernel(static, dynamic, W_s, W_d, b, v, station_num_lim):
    del dynamic
    static2 = static[0]
    wd_col = W_d.astype(f32)
    b_col = b[:, None]
    v_row = v[None, :]
    lim1 = jnp.asarray(station_num_lim, i32).reshape(1)
    s0, s1, st0 = _actor_scores(static2, W_s, wd_col, b_col, v_row, lim1)
    ptr100, logp100 = _decode(s0[0], s1[0], st0[0])
    return ptr100[None, :], logp100[None, :]
